# initial kernel scaffold (unmeasured)
import jax
import jax.numpy as jnp
from jax import lax
from jax.experimental import pallas as pl
from jax.experimental.pallas import tpu as pltpu

N_DEV = 32

_DevId = getattr(pl, "DeviceIdType", None) or pltpu.DeviceIdType
_sem_signal = getattr(pl, "semaphore_signal", None) or pltpu.semaphore_signal
_sem_wait = getattr(pl, "semaphore_wait", None) or pltpu.semaphore_wait


def kernel(x, w_mat, scale_x, scale_w):
    m, _k = x.shape
    _k2, n = w_mat.shape
    chunk = m // N_DEV

    def body(x_ref, w_ref, sx_ref, sw_ref, out_ref, recv_buf, send_sems,
             recv_sems, credit_sem):
        my = lax.axis_index("i")
        left = jnp.mod(my - 1, N_DEV)
        right = jnp.mod(my + 1, N_DEV)

        barrier = pltpu.get_barrier_semaphore()
        _sem_signal(barrier, inc=1, device_id=(left,),
                    device_id_type=_DevId.MESH)
        _sem_signal(barrier, inc=1, device_id=(right,),
                    device_id_type=_DevId.MESH)
        _sem_wait(barrier, 2)

        out_ref[...] = jnp.dot(x_ref[...], w_ref[...],
                               preferred_element_type=jnp.float32)

        def rs_step(s, carry):
            slot = jnp.mod(s, 2)
            c_send = jnp.mod(my - s, N_DEV)
            c_recv = jnp.mod(my - s - 1, N_DEV)

            @pl.when(s >= 2)
            def _():
                _sem_wait(credit_sem, 1)

            rdma = pltpu.make_async_remote_copy(
                src_ref=out_ref.at[pl.ds(c_send * chunk, chunk)],
                dst_ref=recv_buf.at[slot],
                send_sem=send_sems.at[slot],
                recv_sem=recv_sems.at[slot],
                device_id=(right,),
                device_id_type=_DevId.MESH,
            )
            rdma.start()
            rdma.wait()

            rows = pl.ds(c_recv * chunk, chunk)
            out_ref[rows, :] = out_ref[rows, :] + recv_buf[slot]
            _sem_signal(credit_sem, inc=1, device_id=(left,),
                        device_id_type=_DevId.MESH)
            return carry

        lax.fori_loop(0, N_DEV - 1, rs_step, 0)

        own = jnp.mod(my + 1, N_DEV)
        scale = sx_ref[0] * sw_ref[0]
        rows = pl.ds(own * chunk, chunk)
        out_ref[rows, :] = jnp.maximum(out_ref[rows, :] * scale, 0.0)

        def ag_step(t, carry):
            u = t + (N_DEV - 1)
            slot = jnp.mod(u, 2)
            c_send = jnp.mod(my + 1 - t, N_DEV)
            c_recv = jnp.mod(my - t, N_DEV)

            _sem_wait(credit_sem, 1)

            send = pltpu.make_async_remote_copy(
                src_ref=out_ref.at[pl.ds(c_send * chunk, chunk)],
                dst_ref=out_ref.at[pl.ds(c_send * chunk, chunk)],
                send_sem=send_sems.at[slot],
                recv_sem=recv_sems.at[slot],
                device_id=(right,),
                device_id_type=_DevId.MESH,
            )
            send.start()
            send.wait_send()

            recv = pltpu.make_async_remote_copy(
                src_ref=out_ref.at[pl.ds(c_recv * chunk, chunk)],
                dst_ref=out_ref.at[pl.ds(c_recv * chunk, chunk)],
                send_sem=send_sems.at[slot],
                recv_sem=recv_sems.at[slot],
                device_id=(left,),
                device_id_type=_DevId.MESH,
            )
            recv.wait_recv()
            _sem_signal(credit_sem, inc=1, device_id=(left,),
                        device_id_type=_DevId.MESH)
            return carry

        lax.fori_loop(0, N_DEV - 1, ag_step, 0)

        _sem_wait(credit_sem, 2)

    out_shape = jax.ShapeDtypeStruct((m, n), jnp.float32)
    return pl.pallas_call(
        body,
        out_shape=out_shape,
        in_specs=[
            pl.BlockSpec(memory_space=pltpu.VMEM),
            pl.BlockSpec(memory_space=pltpu.VMEM),
            pl.BlockSpec(memory_space=pltpu.SMEM),
            pl.BlockSpec(memory_space=pltpu.SMEM),
        ],
        out_specs=pl.BlockSpec(memory_space=pltpu.VMEM),
        scratch_shapes=[
            pltpu.VMEM((2, chunk, n), jnp.float32),
            pltpu.SemaphoreType.DMA((2,)),
            pltpu.SemaphoreType.DMA((2,)),
            pltpu.SemaphoreType.REGULAR,
        ],
        compiler_params=pltpu.CompilerParams(collective_id=0),
    )(x, w_mat, scale_x, scale_w)


# baseline (device time: 861605 ns/iter reference)
import jax
import jax.numpy as jnp
from jax import lax
from jax.experimental import pallas as pl
from jax.experimental.pallas import tpu as pltpu

N_DEV = 32

_DevId = getattr(pl, "DeviceIdType", None) or pltpu.DeviceIdType
_sem_signal = getattr(pl, "semaphore_signal", None) or pltpu.semaphore_signal
_sem_wait = getattr(pl, "semaphore_wait", None) or pltpu.semaphore_wait


def kernel(x, w_mat, scale_x, scale_w):
    m, _k = x.shape
    _k2, n = w_mat.shape
    chunk = m // N_DEV

    def body(x_ref, w_ref, sx_ref, sw_ref, out_ref, recv_buf, send_sems,
             recv_sems, credit_sem):
        my = lax.axis_index("i")
        left = jnp.mod(my - 1, N_DEV)
        right = jnp.mod(my + 1, N_DEV)

        barrier = pltpu.get_barrier_semaphore()
        _sem_signal(barrier, inc=1, device_id=(left,),
                    device_id_type=_DevId.MESH)
        _sem_signal(barrier, inc=1, device_id=(right,),
                    device_id_type=_DevId.MESH)
        _sem_wait(barrier, 2)

        out_ref[...] = jnp.dot(x_ref[...], w_ref[...],
                               preferred_element_type=jnp.float32)

        def rs_step(s, carry):
            slot = jnp.mod(s, 2)
            c_send = jnp.mod(my - s, N_DEV)
            c_recv = jnp.mod(my - s - 1, N_DEV)

            @pl.when(s >= 2)
            def _():
                _sem_wait(credit_sem, 1)

            rdma = pltpu.make_async_remote_copy(
                src_ref=out_ref.at[pl.ds(c_send * chunk, chunk)],
                dst_ref=recv_buf.at[slot],
                send_sem=send_sems.at[slot],
                recv_sem=recv_sems.at[slot],
                device_id=(right,),
                device_id_type=_DevId.MESH,
            )
            rdma.start()
            rdma.wait()

            rows = pl.ds(c_recv * chunk, chunk)
            out_ref[rows, :] = out_ref[rows, :] + recv_buf[slot]
            _sem_signal(credit_sem, inc=1, device_id=(left,),
                        device_id_type=_DevId.MESH)
            return carry

        lax.fori_loop(0, N_DEV - 1, rs_step, 0)

        own = jnp.mod(my + 1, N_DEV)
        scale = sx_ref[0] * sw_ref[0]
        rows = pl.ds(own * chunk, chunk)
        out_ref[rows, :] = jnp.maximum(out_ref[rows, :] * scale, 0.0)

        def ag_step(t, carry):
            u = t + (N_DEV - 1)
            slot = jnp.mod(u, 2)
            c_send = jnp.mod(my + 1 - t, N_DEV)
            c_recv = jnp.mod(my - t, N_DEV)

            _sem_wait(credit_sem, 1)

            send = pltpu.make_async_remote_copy(
                src_ref=out_ref.at[pl.ds(c_send * chunk, chunk)],
                dst_ref=out_ref.at[pl.ds(c_send * chunk, chunk)],
                send_sem=send_sems.at[slot],
                recv_sem=recv_sems.at[slot],
                device_id=(right,),
                device_id_type=_DevId.MESH,
            )
            send.start()
            send.wait_send()

            recv = pltpu.make_async_remote_copy(
                src_ref=out_ref.at[pl.ds(c_recv * chunk, chunk)],
                dst_ref=out_ref.at[pl.ds(c_recv * chunk, chunk)],
                send_sem=send_sems.at[slot],
                recv_sem=recv_sems.at[slot],
                device_id=(left,),
                device_id_type=_DevId.MESH,
            )
            recv.wait_recv()
            _sem_signal(credit_sem, inc=1, device_id=(left,),
                        device_id_type=_DevId.MESH)
            return carry

        lax.fori_loop(0, N_DEV - 1, ag_step, 0)

        _sem_wait(credit_sem, 2)

    out_shape = jax.ShapeDtypeStruct((m, n), jnp.float32)
    return pl.pallas_call(
        body,
        out_shape=out_shape,
        in_specs=[
            pl.BlockSpec(memory_space=pltpu.VMEM),
            pl.BlockSpec(memory_space=pltpu.VMEM),
            pl.BlockSpec(memory_space=pltpu.SMEM),
            pl.BlockSpec(memory_space=pltpu.SMEM),
        ],
        out_specs=pl.BlockSpec(memory_space=pltpu.VMEM),
        scratch_shapes=[
            pltpu.VMEM((2, chunk, n), jnp.float32),
            pltpu.SemaphoreType.DMA((2,)),
            pltpu.SemaphoreType.DMA((2,)),
            pltpu.SemaphoreType.REGULAR,
        ],
        compiler_params=pltpu.CompilerParams(
            collective_id=0, vmem_limit_bytes=64 * 1024 * 1024),
    )(x, w_mat, scale_x, scale_w)


# device time: 847016 ns/iter; 1.0172x vs baseline; 1.0172x over previous
import jax
import jax.numpy as jnp
from jax import lax
from jax.experimental import pallas as pl
from jax.experimental.pallas import tpu as pltpu

N_DEV = 32

_DevId = getattr(pl, "DeviceIdType", None) or pltpu.DeviceIdType
_sem_signal = getattr(pl, "semaphore_signal", None) or pltpu.semaphore_signal
_sem_wait = getattr(pl, "semaphore_wait", None) or pltpu.semaphore_wait


def kernel(x, w_mat, scale_x, scale_w):
    m, _k = x.shape
    _k2, n = w_mat.shape
    half = m // 2
    ch = half // N_DEV

    def body(x_ref, w_ref, sx_ref, sw_ref, out_ref,
             rbuf_cw, rbuf_ccw, ssem_cw, rsem_cw, ssem_ccw, rsem_ccw,
             credit_cw, credit_ccw):
        my = lax.axis_index("i")
        left = jnp.mod(my - 1, N_DEV)
        right = jnp.mod(my + 1, N_DEV)

        barrier = pltpu.get_barrier_semaphore()
        _sem_signal(barrier, inc=1, device_id=(left,),
                    device_id_type=_DevId.MESH)
        _sem_signal(barrier, inc=1, device_id=(right,),
                    device_id_type=_DevId.MESH)
        _sem_wait(barrier, 2)

        out_ref[...] = jnp.dot(x_ref[...], w_ref[...],
                               preferred_element_type=jnp.float32)

        def cw_rows(c):
            return pl.ds(c * ch, ch)

        def ccw_rows(c):
            return pl.ds(half + c * ch, ch)

        def rs_step(s, carry):
            slot = jnp.mod(s, 2)
            cs_cw = jnp.mod(my - s, N_DEV)
            cr_cw = jnp.mod(my - s - 1, N_DEV)
            cs_ccw = jnp.mod(my + s, N_DEV)
            cr_ccw = jnp.mod(my + s + 1, N_DEV)

            @pl.when(s >= 2)
            def _():
                _sem_wait(credit_cw, 1)
                _sem_wait(credit_ccw, 1)

            rd_cw = pltpu.make_async_remote_copy(
                src_ref=out_ref.at[cw_rows(cs_cw)],
                dst_ref=rbuf_cw.at[slot],
                send_sem=ssem_cw.at[slot],
                recv_sem=rsem_cw.at[slot],
                device_id=(right,),
                device_id_type=_DevId.MESH,
            )
            rd_ccw = pltpu.make_async_remote_copy(
                src_ref=out_ref.at[ccw_rows(cs_ccw)],
                dst_ref=rbuf_ccw.at[slot],
                send_sem=ssem_ccw.at[slot],
                recv_sem=rsem_ccw.at[slot],
                device_id=(left,),
                device_id_type=_DevId.MESH,
            )
            rd_cw.start()
            rd_ccw.start()

            rd_cw.wait()
            r = cw_rows(cr_cw)
            out_ref[r, :] = out_ref[r, :] + rbuf_cw[slot]
            _sem_signal(credit_cw, inc=1, device_id=(left,),
                        device_id_type=_DevId.MESH)

            rd_ccw.wait()
            r = ccw_rows(cr_ccw)
            out_ref[r, :] = out_ref[r, :] + rbuf_ccw[slot]
            _sem_signal(credit_ccw, inc=1, device_id=(right,),
                        device_id_type=_DevId.MESH)
            return carry

        lax.fori_loop(0, N_DEV - 1, rs_step, 0)

        scale = sx_ref[0] * sw_ref[0]
        own_cw = jnp.mod(my + 1, N_DEV)
        own_ccw = jnp.mod(my - 1, N_DEV)
        r = cw_rows(own_cw)
        out_ref[r, :] = jnp.maximum(out_ref[r, :] * scale, 0.0)
        r = ccw_rows(own_ccw)
        out_ref[r, :] = jnp.maximum(out_ref[r, :] * scale, 0.0)

        def ag_step(t, carry):
            u = t + (N_DEV - 1)
            slot = jnp.mod(u, 2)
            cs_cw = jnp.mod(my + 1 - t, N_DEV)
            cr_cw = jnp.mod(my - t, N_DEV)
            cs_ccw = jnp.mod(my - 1 + t, N_DEV)
            cr_ccw = jnp.mod(my + t, N_DEV)

            _sem_wait(credit_cw, 1)
            _sem_wait(credit_ccw, 1)

            send_cw = pltpu.make_async_remote_copy(
                src_ref=out_ref.at[cw_rows(cs_cw)],
                dst_ref=out_ref.at[cw_rows(cs_cw)],
                send_sem=ssem_cw.at[slot],
                recv_sem=rsem_cw.at[slot],
                device_id=(right,),
                device_id_type=_DevId.MESH,
            )
            send_ccw = pltpu.make_async_remote_copy(
                src_ref=out_ref.at[ccw_rows(cs_ccw)],
                dst_ref=out_ref.at[ccw_rows(cs_ccw)],
                send_sem=ssem_ccw.at[slot],
                recv_sem=rsem_ccw.at[slot],
                device_id=(left,),
                device_id_type=_DevId.MESH,
            )
            send_cw.start()
            send_ccw.start()
            send_cw.wait_send()
            send_ccw.wait_send()

            recv_cw = pltpu.make_async_remote_copy(
                src_ref=out_ref.at[cw_rows(cr_cw)],
                dst_ref=out_ref.at[cw_rows(cr_cw)],
                send_sem=ssem_cw.at[slot],
                recv_sem=rsem_cw.at[slot],
                device_id=(left,),
                device_id_type=_DevId.MESH,
            )
            recv_ccw = pltpu.make_async_remote_copy(
                src_ref=out_ref.at[ccw_rows(cr_ccw)],
                dst_ref=out_ref.at[ccw_rows(cr_ccw)],
                send_sem=ssem_ccw.at[slot],
                recv_sem=rsem_ccw.at[slot],
                device_id=(right,),
                device_id_type=_DevId.MESH,
            )
            recv_cw.wait_recv()
            _sem_signal(credit_cw, inc=1, device_id=(left,),
                        device_id_type=_DevId.MESH)
            recv_ccw.wait_recv()
            _sem_signal(credit_ccw, inc=1, device_id=(right,),
                        device_id_type=_DevId.MESH)
            return carry

        lax.fori_loop(0, N_DEV - 1, ag_step, 0)

        _sem_wait(credit_cw, 2)
        _sem_wait(credit_ccw, 2)

    out_shape = jax.ShapeDtypeStruct((m, n), jnp.float32)
    return pl.pallas_call(
        body,
        out_shape=out_shape,
        in_specs=[
            pl.BlockSpec(memory_space=pltpu.VMEM),
            pl.BlockSpec(memory_space=pltpu.VMEM),
            pl.BlockSpec(memory_space=pltpu.SMEM),
            pl.BlockSpec(memory_space=pltpu.SMEM),
        ],
        out_specs=pl.BlockSpec(memory_space=pltpu.VMEM),
        scratch_shapes=[
            pltpu.VMEM((2, ch, n), jnp.float32),
            pltpu.VMEM((2, ch, n), jnp.float32),
            pltpu.SemaphoreType.DMA((2,)),
            pltpu.SemaphoreType.DMA((2,)),
            pltpu.SemaphoreType.DMA((2,)),
            pltpu.SemaphoreType.DMA((2,)),
            pltpu.SemaphoreType.REGULAR,
            pltpu.SemaphoreType.REGULAR,
        ],
        compiler_params=pltpu.CompilerParams(
            collective_id=0, vmem_limit_bytes=64 * 1024 * 1024),
    )(x, w_mat, scale_x, scale_w)


# device time: 508590 ns/iter; 1.6941x vs baseline; 1.6654x over previous
import jax
import jax.numpy as jnp
from jax import lax
from jax.experimental import pallas as pl
from jax.experimental.pallas import tpu as pltpu

N_DEV = 32

_DevId = getattr(pl, "DeviceIdType", None) or pltpu.DeviceIdType
_sem_signal = getattr(pl, "semaphore_signal", None) or pltpu.semaphore_signal
_sem_wait = getattr(pl, "semaphore_wait", None) or pltpu.semaphore_wait

RING_OF_LOG = (0, 31, 30, 1, 2, 29, 28, 3, 7, 24, 25, 6, 5, 26, 27, 4,
               8, 23, 22, 9, 10, 21, 20, 11, 15, 16, 17, 14, 13, 18, 19, 12)
LOG_OF_RING = (0, 3, 4, 7, 15, 12, 11, 8, 16, 19, 20, 23, 31, 28, 27, 24,
               25, 26, 29, 30, 22, 21, 18, 17, 9, 10, 13, 14, 6, 5, 2, 1)


def kernel(x, w_mat, scale_x, scale_w):
    m, _k = x.shape
    _k2, n = w_mat.shape
    half = m // 2
    ch = half // N_DEV

    def body(x_ref, w_ref, sx_ref, sw_ref, pos_ref, out_ref,
             rbuf_cw, rbuf_ccw, ssem_cw, rsem_cw, ssem_ccw, rsem_ccw,
             credit_cw, credit_ccw):
        my = pos_ref[0]
        left = pos_ref[1]
        right = pos_ref[2]

        barrier = pltpu.get_barrier_semaphore()
        _sem_signal(barrier, inc=1, device_id=(left,),
                    device_id_type=_DevId.MESH)
        _sem_signal(barrier, inc=1, device_id=(right,),
                    device_id_type=_DevId.MESH)
        _sem_wait(barrier, 2)

        out_ref[...] = jnp.dot(x_ref[...], w_ref[...],
                               preferred_element_type=jnp.float32)

        def cw_rows(c):
            return pl.ds(c * ch, ch)

        def ccw_rows(c):
            return pl.ds(half + c * ch, ch)

        def rs_step(s, carry):
            slot = jnp.mod(s, 2)
            cs_cw = jnp.mod(my - s, N_DEV)
            cr_cw = jnp.mod(my - s - 1, N_DEV)
            cs_ccw = jnp.mod(my + s, N_DEV)
            cr_ccw = jnp.mod(my + s + 1, N_DEV)

            @pl.when(s >= 2)
            def _():
                _sem_wait(credit_cw, 1)
                _sem_wait(credit_ccw, 1)

            rd_cw = pltpu.make_async_remote_copy(
                src_ref=out_ref.at[cw_rows(cs_cw)],
                dst_ref=rbuf_cw.at[slot],
                send_sem=ssem_cw.at[slot],
                recv_sem=rsem_cw.at[slot],
                device_id=(right,),
                device_id_type=_DevId.MESH,
            )
            rd_ccw = pltpu.make_async_remote_copy(
                src_ref=out_ref.at[ccw_rows(cs_ccw)],
                dst_ref=rbuf_ccw.at[slot],
                send_sem=ssem_ccw.at[slot],
                recv_sem=rsem_ccw.at[slot],
                device_id=(left,),
                device_id_type=_DevId.MESH,
            )
            rd_cw.start()
            rd_ccw.start()

            rd_cw.wait()
            r = cw_rows(cr_cw)
            out_ref[r, :] = out_ref[r, :] + rbuf_cw[slot]
            _sem_signal(credit_cw, inc=1, device_id=(left,),
                        device_id_type=_DevId.MESH)

            rd_ccw.wait()
            r = ccw_rows(cr_ccw)
            out_ref[r, :] = out_ref[r, :] + rbuf_ccw[slot]
            _sem_signal(credit_ccw, inc=1, device_id=(right,),
                        device_id_type=_DevId.MESH)
            return carry

        lax.fori_loop(0, N_DEV - 1, rs_step, 0)

        scale = sx_ref[0] * sw_ref[0]
        own_cw = jnp.mod(my + 1, N_DEV)
        own_ccw = jnp.mod(my - 1, N_DEV)
        r = cw_rows(own_cw)
        out_ref[r, :] = jnp.maximum(out_ref[r, :] * scale, 0.0)
        r = ccw_rows(own_ccw)
        out_ref[r, :] = jnp.maximum(out_ref[r, :] * scale, 0.0)

        def ag_step(t, carry):
            u = t + (N_DEV - 1)
            slot = jnp.mod(u, 2)
            cs_cw = jnp.mod(my + 1 - t, N_DEV)
            cr_cw = jnp.mod(my - t, N_DEV)
            cs_ccw = jnp.mod(my - 1 + t, N_DEV)
            cr_ccw = jnp.mod(my + t, N_DEV)

            _sem_wait(credit_cw, 1)
            _sem_wait(credit_ccw, 1)

            send_cw = pltpu.make_async_remote_copy(
                src_ref=out_ref.at[cw_rows(cs_cw)],
                dst_ref=out_ref.at[cw_rows(cs_cw)],
                send_sem=ssem_cw.at[slot],
                recv_sem=rsem_cw.at[slot],
                device_id=(right,),
                device_id_type=_DevId.MESH,
            )
            send_ccw = pltpu.make_async_remote_copy(
                src_ref=out_ref.at[ccw_rows(cs_ccw)],
                dst_ref=out_ref.at[ccw_rows(cs_ccw)],
                send_sem=ssem_ccw.at[slot],
                recv_sem=rsem_ccw.at[slot],
                device_id=(left,),
                device_id_type=_DevId.MESH,
            )
            send_cw.start()
            send_ccw.start()
            send_cw.wait_send()
            send_ccw.wait_send()

            recv_cw = pltpu.make_async_remote_copy(
                src_ref=out_ref.at[cw_rows(cr_cw)],
                dst_ref=out_ref.at[cw_rows(cr_cw)],
                send_sem=ssem_cw.at[slot],
                recv_sem=rsem_cw.at[slot],
                device_id=(left,),
                device_id_type=_DevId.MESH,
            )
            recv_ccw = pltpu.make_async_remote_copy(
                src_ref=out_ref.at[ccw_rows(cr_ccw)],
                dst_ref=out_ref.at[ccw_rows(cr_ccw)],
                send_sem=ssem_ccw.at[slot],
                recv_sem=rsem_ccw.at[slot],
                device_id=(right,),
                device_id_type=_DevId.MESH,
            )
            recv_cw.wait_recv()
            _sem_signal(credit_cw, inc=1, device_id=(left,),
                        device_id_type=_DevId.MESH)
            recv_ccw.wait_recv()
            _sem_signal(credit_ccw, inc=1, device_id=(right,),
                        device_id_type=_DevId.MESH)
            return carry

        lax.fori_loop(0, N_DEV - 1, ag_step, 0)

        _sem_wait(credit_cw, 2)
        _sem_wait(credit_ccw, 2)

    my_log = lax.axis_index("i")
    ring_of_log = jnp.asarray(RING_OF_LOG, dtype=jnp.int32)
    log_of_ring = jnp.asarray(LOG_OF_RING, dtype=jnp.int32)
    rp = ring_of_log[my_log]
    left_log = log_of_ring[jnp.mod(rp - 1, N_DEV)]
    right_log = log_of_ring[jnp.mod(rp + 1, N_DEV)]
    pos = jnp.stack([rp, left_log, right_log]).astype(jnp.int32)

    out_shape = jax.ShapeDtypeStruct((m, n), jnp.float32)
    return pl.pallas_call(
        body,
        out_shape=out_shape,
        in_specs=[
            pl.BlockSpec(memory_space=pltpu.VMEM),
            pl.BlockSpec(memory_space=pltpu.VMEM),
            pl.BlockSpec(memory_space=pltpu.SMEM),
            pl.BlockSpec(memory_space=pltpu.SMEM),
            pl.BlockSpec(memory_space=pltpu.SMEM),
        ],
        out_specs=pl.BlockSpec(memory_space=pltpu.VMEM),
        scratch_shapes=[
            pltpu.VMEM((2, ch, n), jnp.float32),
            pltpu.VMEM((2, ch, n), jnp.float32),
            pltpu.SemaphoreType.DMA((2,)),
            pltpu.SemaphoreType.DMA((2,)),
            pltpu.SemaphoreType.DMA((2,)),
            pltpu.SemaphoreType.DMA((2,)),
            pltpu.SemaphoreType.REGULAR,
            pltpu.SemaphoreType.REGULAR,
        ],
        compiler_params=pltpu.CompilerParams(
            collective_id=0, vmem_limit_bytes=64 * 1024 * 1024),
    )(x, w_mat, scale_x, scale_w, pos)


# device time: 404103 ns/iter; 2.1321x vs baseline; 1.2586x over previous
import jax
import jax.numpy as jnp
from jax import lax
from jax.experimental import pallas as pl
from jax.experimental.pallas import tpu as pltpu

N_DEV = 32

_DevId = getattr(pl, "DeviceIdType", None) or pltpu.DeviceIdType
_sem_signal = getattr(pl, "semaphore_signal", None) or pltpu.semaphore_signal
_sem_wait = getattr(pl, "semaphore_wait", None) or pltpu.semaphore_wait

RING_OF_LOG = (0, 31, 30, 1, 2, 29, 28, 3, 7, 24, 25, 6, 5, 26, 27, 4,
               8, 23, 22, 9, 10, 21, 20, 11, 15, 16, 17, 14, 13, 18, 19, 12)
LOG_OF_RING = (0, 3, 4, 7, 15, 12, 11, 8, 16, 19, 20, 23, 31, 28, 27, 24,
               25, 26, 29, 30, 22, 21, 18, 17, 9, 10, 13, 14, 6, 5, 2, 1)


def kernel(x, w_mat, scale_x, scale_w):
    m, _k = x.shape
    _k2, n = w_mat.shape
    half = m // 2
    ch = half // N_DEV
    sub = ch // 2

    def body(x_ref, w_ref, sx_ref, sw_ref, pos_ref, out_ref,
             rb0, rb1, rb2, rb3, ss0, rs0, ss1, rs1, ss2, rs2, ss3, rs3,
             cr0, cr1, cr2, cr3):
        my = pos_ref[0]
        left = pos_ref[1]
        right = pos_ref[2]

        barrier = pltpu.get_barrier_semaphore()
        _sem_signal(barrier, inc=1, device_id=(left,),
                    device_id_type=_DevId.MESH)
        _sem_signal(barrier, inc=1, device_id=(right,),
                    device_id_type=_DevId.MESH)
        _sem_wait(barrier, 2)

        out_ref[...] = jnp.dot(x_ref[...], w_ref[...],
                               preferred_element_type=jnp.float32)

        streams = [
            dict(rb=rb0, ss=ss0, rs=rs0, cr=cr0, d=0, h=0),
            dict(rb=rb2, ss=ss2, rs=rs2, cr=cr2, d=1, h=0),
            dict(rb=rb1, ss=ss1, rs=rs1, cr=cr1, d=0, h=1),
            dict(rb=rb3, ss=ss3, rs=rs3, cr=cr3, d=1, h=1),
        ]
        for st in streams:
            st["dev"] = right if st["d"] == 0 else left
            st["updev"] = left if st["d"] == 0 else right

        def rows(st, c):
            return pl.ds(st["d"] * half + c * ch + st["h"] * sub, sub)

        def cs_rs(st, s):
            return jnp.mod(my - s if st["d"] == 0 else my + s, N_DEV)

        def cr_rs(st, s):
            return jnp.mod(my - s - 1 if st["d"] == 0 else my + s + 1, N_DEV)

        def cs_ag(st, t):
            return jnp.mod(my + 1 - t if st["d"] == 0 else my - 1 + t, N_DEV)

        def cr_ag(st, t):
            return jnp.mod(my - t if st["d"] == 0 else my + t, N_DEV)

        def rs_desc(st, s):
            slot = jnp.mod(s, 2)
            return pltpu.make_async_remote_copy(
                src_ref=out_ref.at[rows(st, cs_rs(st, s))],
                dst_ref=st["rb"].at[slot],
                send_sem=st["ss"].at[slot],
                recv_sem=st["rs"].at[slot],
                device_id=(st["dev"],),
                device_id_type=_DevId.MESH,
            )

        def ag_send_desc(st, t):
            slot = jnp.mod(t + N_DEV - 1, 2)
            r = rows(st, cs_ag(st, t))
            return pltpu.make_async_remote_copy(
                src_ref=out_ref.at[r],
                dst_ref=out_ref.at[r],
                send_sem=st["ss"].at[slot],
                recv_sem=st["rs"].at[slot],
                device_id=(st["dev"],),
                device_id_type=_DevId.MESH,
            )

        def ag_recv_desc(st, t):
            slot = jnp.mod(t + N_DEV - 1, 2)
            r = rows(st, cr_ag(st, t))
            return pltpu.make_async_remote_copy(
                src_ref=out_ref.at[r],
                dst_ref=out_ref.at[r],
                send_sem=st["ss"].at[slot],
                recv_sem=st["rs"].at[slot],
                device_id=(st["updev"],),
                device_id_type=_DevId.MESH,
            )

        for st in streams:
            rs_desc(st, 0).start()

        def rs_iter(s, carry):
            for st in streams:
                rs_desc(st, s).wait_recv()
                r = rows(st, cr_rs(st, s))
                out_ref[r, :] = out_ref[r, :] + st["rb"][jnp.mod(s, 2)]
                _sem_signal(st["cr"], inc=1, device_id=(st["updev"],),
                            device_id_type=_DevId.MESH)

                @pl.when(s <= N_DEV - 3)
                def _(st=st, s=s):
                    @pl.when(s >= 1)
                    def _(st=st, s=s):
                        _sem_wait(st["cr"], 1)
                        rs_desc(st, s - 1).wait_send()
                    rs_desc(st, s + 1).start()
            return carry

        lax.fori_loop(0, N_DEV - 1, rs_iter, 0)

        scale = sx_ref[0] * sw_ref[0]
        for st in streams:
            own = jnp.mod(my + 1 if st["d"] == 0 else my - 1, N_DEV)
            r = rows(st, own)
            out_ref[r, :] = jnp.maximum(out_ref[r, :] * scale, 0.0)

        for st in streams:
            _sem_wait(st["cr"], 1)
            rs_desc(st, N_DEV - 3).wait_send()
            ag_send_desc(st, 0).start()

        def ag_iter(t, carry):
            for st in streams:
                ag_recv_desc(st, t).wait_recv()
                _sem_signal(st["cr"], inc=1, device_id=(st["updev"],),
                            device_id_type=_DevId.MESH)

                @pl.when(t <= N_DEV - 3)
                def _(st=st, t=t):
                    _sem_wait(st["cr"], 1)
                    ag_send_desc(st, t - 1).wait_send()
                    ag_send_desc(st, t + 1).start()
            return carry

        lax.fori_loop(0, N_DEV - 1, ag_iter, 0)

        for st in streams:
            ag_send_desc(st, N_DEV - 3).wait_send()
            ag_send_desc(st, N_DEV - 2).wait_send()
            _sem_wait(st["cr"], 2)

    my_log = lax.axis_index("i")
    ring_of_log = jnp.asarray(RING_OF_LOG, dtype=jnp.int32)
    log_of_ring = jnp.asarray(LOG_OF_RING, dtype=jnp.int32)
    rp = ring_of_log[my_log]
    left_log = log_of_ring[jnp.mod(rp - 1, N_DEV)]
    right_log = log_of_ring[jnp.mod(rp + 1, N_DEV)]
    pos = jnp.stack([rp, left_log, right_log]).astype(jnp.int32)

    out_shape = jax.ShapeDtypeStruct((m, n), jnp.float32)
    return pl.pallas_call(
        body,
        out_shape=out_shape,
        in_specs=[
            pl.BlockSpec(memory_space=pltpu.VMEM),
            pl.BlockSpec(memory_space=pltpu.VMEM),
            pl.BlockSpec(memory_space=pltpu.SMEM),
            pl.BlockSpec(memory_space=pltpu.SMEM),
            pl.BlockSpec(memory_space=pltpu.SMEM),
        ],
        out_specs=pl.BlockSpec(memory_space=pltpu.VMEM),
        scratch_shapes=[
            pltpu.VMEM((2, sub, n), jnp.float32),
            pltpu.VMEM((2, sub, n), jnp.float32),
            pltpu.VMEM((2, sub, n), jnp.float32),
            pltpu.VMEM((2, sub, n), jnp.float32),
            pltpu.SemaphoreType.DMA((2,)),
            pltpu.SemaphoreType.DMA((2,)),
            pltpu.SemaphoreType.DMA((2,)),
            pltpu.SemaphoreType.DMA((2,)),
            pltpu.SemaphoreType.DMA((2,)),
            pltpu.SemaphoreType.DMA((2,)),
            pltpu.SemaphoreType.DMA((2,)),
            pltpu.SemaphoreType.DMA((2,)),
            pltpu.SemaphoreType.REGULAR,
            pltpu.SemaphoreType.REGULAR,
            pltpu.SemaphoreType.REGULAR,
            pltpu.SemaphoreType.REGULAR,
        ],
        compiler_params=pltpu.CompilerParams(
            collective_id=0, vmem_limit_bytes=64 * 1024 * 1024),
    )(x, w_mat, scale_x, scale_w, pos)


# device time: 249412 ns/iter; 3.4545x vs baseline; 1.6202x over previous
import jax
import jax.numpy as jnp
from jax import lax
from jax.experimental import pallas as pl
from jax.experimental.pallas import tpu as pltpu

N_DEV = 32

_DevId = getattr(pl, "DeviceIdType", None) or pltpu.DeviceIdType
_sem_signal = getattr(pl, "semaphore_signal", None) or pltpu.semaphore_signal
_sem_wait = getattr(pl, "semaphore_wait", None) or pltpu.semaphore_wait

RING_OF_LOG = (0, 31, 30, 1, 2, 29, 28, 3, 7, 24, 25, 6, 5, 26, 27, 4,
               8, 23, 22, 9, 10, 21, 20, 11, 15, 16, 17, 14, 13, 18, 19, 12)
LOG_OF_RING = (0, 3, 4, 7, 15, 12, 11, 8, 16, 19, 20, 23, 31, 28, 27, 24,
               25, 26, 29, 30, 22, 21, 18, 17, 9, 10, 13, 14, 6, 5, 2, 1)


def kernel(x, w_mat, scale_x, scale_w):
    m, _k = x.shape
    _k2, n = w_mat.shape
    half = m // 2
    ch = half // N_DEV
    sub = ch // 2

    def body(x_ref, w_ref, sx_ref, sw_ref, pos_ref, out_ref,
             rb0, rb1, rb2, rb3, sb0, sb1, sb2, sb3,
             ss0, rs0, ss1, rs1, ss2, rs2, ss3, rs3,
             cr0, cr1, cr2, cr3):
        my = pos_ref[0]
        left = pos_ref[1]
        right = pos_ref[2]

        barrier = pltpu.get_barrier_semaphore()
        _sem_signal(barrier, inc=1, device_id=(left,),
                    device_id_type=_DevId.MESH)
        _sem_signal(barrier, inc=1, device_id=(right,),
                    device_id_type=_DevId.MESH)
        _sem_wait(barrier, 2)

        out_ref[...] = jnp.dot(x_ref[...], w_ref[...],
                               preferred_element_type=jnp.float32)

        streams = [
            dict(rb=rb0, sb=sb0, ss=ss0, rs=rs0, cr=cr0, d=0, h=0),
            dict(rb=rb2, sb=sb2, ss=ss2, rs=rs2, cr=cr2, d=1, h=0),
            dict(rb=rb1, sb=sb1, ss=ss1, rs=rs1, cr=cr1, d=0, h=1),
            dict(rb=rb3, sb=sb3, ss=ss3, rs=rs3, cr=cr3, d=1, h=1),
        ]
        for st in streams:
            st["dev"] = right if st["d"] == 0 else left
            st["updev"] = left if st["d"] == 0 else right

        def rows(st, c):
            return pl.ds(st["d"] * half + c * ch + st["h"] * sub, sub)

        def cs_rs(st, s):
            return jnp.mod(my - s if st["d"] == 0 else my + s, N_DEV)

        def cr_rs(st, s):
            return jnp.mod(my - s - 1 if st["d"] == 0 else my + s + 1, N_DEV)

        def cs_ag(st, t):
            return jnp.mod(my + 1 - t if st["d"] == 0 else my - 1 + t, N_DEV)

        def cr_ag(st, t):
            return jnp.mod(my - t if st["d"] == 0 else my + t, N_DEV)

        def rs_desc(st, s):
            slot = jnp.mod(s, 2)
            return pltpu.make_async_remote_copy(
                src_ref=st["sb"].at[slot],
                dst_ref=st["rb"].at[slot],
                send_sem=st["ss"].at[slot],
                recv_sem=st["rs"].at[slot],
                device_id=(st["dev"],),
                device_id_type=_DevId.MESH,
            )

        def ag_desc(st, k):
            slot = jnp.mod(k + N_DEV - 1, 2)
            return pltpu.make_async_remote_copy(
                src_ref=st["sb"].at[slot],
                dst_ref=st["rb"].at[slot],
                send_sem=st["ss"].at[slot],
                recv_sem=st["rs"].at[slot],
                device_id=(st["dev"],),
                device_id_type=_DevId.MESH,
            )

        def rs_stage_start(st, s):
            slot = jnp.mod(s, 2)
            st["sb"][slot] = out_ref[rows(st, cs_rs(st, s)), :].astype(
                jnp.bfloat16)
            rs_desc(st, s).start()

        def ag_stage_start(st, k):
            slot = jnp.mod(k + N_DEV - 1, 2)
            st["sb"][slot] = out_ref[rows(st, cs_ag(st, k)), :].astype(
                jnp.bfloat16)
            ag_desc(st, k).start()

        for st in streams:
            rs_stage_start(st, 0)

        def rs_iter(s, carry):
            for st in streams:
                rs_desc(st, s).wait_recv()
                r = rows(st, cr_rs(st, s))
                out_ref[r, :] = out_ref[r, :] + st["rb"][jnp.mod(s, 2)].astype(
                    jnp.float32)
                _sem_signal(st["cr"], inc=1, device_id=(st["updev"],),
                            device_id_type=_DevId.MESH)

                @pl.when(s <= N_DEV - 3)
                def _(st=st, s=s):
                    @pl.when(s >= 1)
                    def _(st=st, s=s):
                        _sem_wait(st["cr"], 1)
                        rs_desc(st, s - 1).wait_send()
                    rs_stage_start(st, s + 1)
            return carry

        lax.fori_loop(0, N_DEV - 1, rs_iter, 0)

        scale = sx_ref[0] * sw_ref[0]
        for st in streams:
            own = jnp.mod(my + 1 if st["d"] == 0 else my - 1, N_DEV)
            r = rows(st, own)
            out_ref[r, :] = jnp.maximum(out_ref[r, :] * scale, 0.0)

        for st in streams:
            _sem_wait(st["cr"], 1)
            rs_desc(st, N_DEV - 3).wait_send()
            ag_stage_start(st, 0)

        def ag_iter(t, carry):
            for st in streams:
                ag_desc(st, t).wait_recv()
                slot = jnp.mod(t + N_DEV - 1, 2)
                r = rows(st, cr_ag(st, t))
                out_ref[r, :] = st["rb"][slot].astype(jnp.float32)
                _sem_signal(st["cr"], inc=1, device_id=(st["updev"],),
                            device_id_type=_DevId.MESH)

                @pl.when(t <= N_DEV - 3)
                def _(st=st, t=t):
                    _sem_wait(st["cr"], 1)
                    ag_desc(st, t - 1).wait_send()
                    ag_stage_start(st, t + 1)
            return carry

        lax.fori_loop(0, N_DEV - 1, ag_iter, 0)

        for st in streams:
            ag_desc(st, N_DEV - 3).wait_send()
            ag_desc(st, N_DEV - 2).wait_send()
            _sem_wait(st["cr"], 2)

    my_log = lax.axis_index("i")
    ring_of_log = jnp.asarray(RING_OF_LOG, dtype=jnp.int32)
    log_of_ring = jnp.asarray(LOG_OF_RING, dtype=jnp.int32)
    rp = ring_of_log[my_log]
    left_log = log_of_ring[jnp.mod(rp - 1, N_DEV)]
    right_log = log_of_ring[jnp.mod(rp + 1, N_DEV)]
    pos = jnp.stack([rp, left_log, right_log]).astype(jnp.int32)

    out_shape = jax.ShapeDtypeStruct((m, n), jnp.float32)
    return pl.pallas_call(
        body,
        out_shape=out_shape,
        in_specs=[
            pl.BlockSpec(memory_space=pltpu.VMEM),
            pl.BlockSpec(memory_space=pltpu.VMEM),
            pl.BlockSpec(memory_space=pltpu.SMEM),
            pl.BlockSpec(memory_space=pltpu.SMEM),
            pl.BlockSpec(memory_space=pltpu.SMEM),
        ],
        out_specs=pl.BlockSpec(memory_space=pltpu.VMEM),
        scratch_shapes=[
            pltpu.VMEM((2, sub, n), jnp.bfloat16),
            pltpu.VMEM((2, sub, n), jnp.bfloat16),
            pltpu.VMEM((2, sub, n), jnp.bfloat16),
            pltpu.VMEM((2, sub, n), jnp.bfloat16),
            pltpu.VMEM((2, sub, n), jnp.bfloat16),
            pltpu.VMEM((2, sub, n), jnp.bfloat16),
            pltpu.VMEM((2, sub, n), jnp.bfloat16),
            pltpu.VMEM((2, sub, n), jnp.bfloat16),
            pltpu.SemaphoreType.DMA((2,)),
            pltpu.SemaphoreType.DMA((2,)),
            pltpu.SemaphoreType.DMA((2,)),
            pltpu.SemaphoreType.DMA((2,)),
            pltpu.SemaphoreType.DMA((2,)),
            pltpu.SemaphoreType.DMA((2,)),
            pltpu.SemaphoreType.DMA((2,)),
            pltpu.SemaphoreType.DMA((2,)),
            pltpu.SemaphoreType.REGULAR,
            pltpu.SemaphoreType.REGULAR,
            pltpu.SemaphoreType.REGULAR,
            pltpu.SemaphoreType.REGULAR,
        ],
        compiler_params=pltpu.CompilerParams(
            collective_id=0, vmem_limit_bytes=64 * 1024 * 1024),
    )(x, w_mat, scale_x, scale_w, pos)


# device time: 249388 ns/iter; 3.4549x vs baseline; 1.0001x over previous
import jax
import jax.numpy as jnp
from jax import lax
from jax.experimental import pallas as pl
from jax.experimental.pallas import tpu as pltpu

N_DEV = 32

_DevId = getattr(pl, "DeviceIdType", None) or pltpu.DeviceIdType
_sem_signal = getattr(pl, "semaphore_signal", None) or pltpu.semaphore_signal
_sem_wait = getattr(pl, "semaphore_wait", None) or pltpu.semaphore_wait

RING_OF_LOG = (0, 31, 30, 1, 2, 29, 28, 3, 7, 24, 25, 6, 5, 26, 27, 4,
               8, 23, 22, 9, 10, 21, 20, 11, 15, 16, 17, 14, 13, 18, 19, 12)
LOG_OF_RING = (0, 3, 4, 7, 15, 12, 11, 8, 16, 19, 20, 23, 31, 28, 27, 24,
               25, 26, 29, 30, 22, 21, 18, 17, 9, 10, 13, 14, 6, 5, 2, 1)


def kernel(x, w_mat, scale_x, scale_w):
    m, _k = x.shape
    _k2, n = w_mat.shape
    half = m // 2
    ch = half // N_DEV
    sub = ch // 2

    def body(x_ref, w_ref, sx_ref, sw_ref, pos_ref, out_ref,
             rb0, rb1, rb2, rb3, sb0, sb1, sb2, sb3,
             ss0, rs0, ss1, rs1, ss2, rs2, ss3, rs3,
             cr0, cr1, cr2, cr3):
        my = pos_ref[0]
        left = pos_ref[1]
        right = pos_ref[2]

        barrier = pltpu.get_barrier_semaphore()
        _sem_signal(barrier, inc=1, device_id=(left,),
                    device_id_type=_DevId.MESH)
        _sem_signal(barrier, inc=1, device_id=(right,),
                    device_id_type=_DevId.MESH)
        _sem_wait(barrier, 2)

        out_ref[...] = jnp.dot(x_ref[...].astype(jnp.bfloat16),
                               w_ref[...].astype(jnp.bfloat16),
                               preferred_element_type=jnp.float32)

        streams = [
            dict(rb=rb0, sb=sb0, ss=ss0, rs=rs0, cr=cr0, d=0, h=0),
            dict(rb=rb2, sb=sb2, ss=ss2, rs=rs2, cr=cr2, d=1, h=0),
            dict(rb=rb1, sb=sb1, ss=ss1, rs=rs1, cr=cr1, d=0, h=1),
            dict(rb=rb3, sb=sb3, ss=ss3, rs=rs3, cr=cr3, d=1, h=1),
        ]
        for st in streams:
            st["dev"] = right if st["d"] == 0 else left
            st["updev"] = left if st["d"] == 0 else right

        def rows(st, c):
            return pl.ds(st["d"] * half + c * ch + st["h"] * sub, sub)

        def cs_rs(st, s):
            return jnp.mod(my - s if st["d"] == 0 else my + s, N_DEV)

        def cr_rs(st, s):
            return jnp.mod(my - s - 1 if st["d"] == 0 else my + s + 1, N_DEV)

        def cs_ag(st, t):
            return jnp.mod(my + 1 - t if st["d"] == 0 else my - 1 + t, N_DEV)

        def cr_ag(st, t):
            return jnp.mod(my - t if st["d"] == 0 else my + t, N_DEV)

        def rs_desc(st, s):
            slot = jnp.mod(s, 2)
            return pltpu.make_async_remote_copy(
                src_ref=st["sb"].at[slot],
                dst_ref=st["rb"].at[slot],
                send_sem=st["ss"].at[slot],
                recv_sem=st["rs"].at[slot],
                device_id=(st["dev"],),
                device_id_type=_DevId.MESH,
            )

        def ag_desc(st, k):
            slot = jnp.mod(k + N_DEV - 1, 2)
            return pltpu.make_async_remote_copy(
                src_ref=st["sb"].at[slot],
                dst_ref=st["rb"].at[slot],
                send_sem=st["ss"].at[slot],
                recv_sem=st["rs"].at[slot],
                device_id=(st["dev"],),
                device_id_type=_DevId.MESH,
            )

        def rs_stage_start(st, s):
            slot = jnp.mod(s, 2)
            st["sb"][slot] = out_ref[rows(st, cs_rs(st, s)), :].astype(
                jnp.bfloat16)
            rs_desc(st, s).start()

        def ag_stage_start(st, k):
            slot = jnp.mod(k + N_DEV - 1, 2)
            st["sb"][slot] = out_ref[rows(st, cs_ag(st, k)), :].astype(
                jnp.bfloat16)
            ag_desc(st, k).start()

        for st in streams:
            rs_stage_start(st, 0)

        def rs_iter(s, carry):
            for st in streams:
                rs_desc(st, s).wait_recv()
                r = rows(st, cr_rs(st, s))
                out_ref[r, :] = out_ref[r, :] + st["rb"][jnp.mod(s, 2)].astype(
                    jnp.float32)
                _sem_signal(st["cr"], inc=1, device_id=(st["updev"],),
                            device_id_type=_DevId.MESH)

                @pl.when(s <= N_DEV - 3)
                def _(st=st, s=s):
                    @pl.when(s >= 1)
                    def _(st=st, s=s):
                        _sem_wait(st["cr"], 1)
                        rs_desc(st, s - 1).wait_send()
                    rs_stage_start(st, s + 1)
            return carry

        lax.fori_loop(0, N_DEV - 1, rs_iter, 0)

        scale = sx_ref[0] * sw_ref[0]
        for st in streams:
            own = jnp.mod(my + 1 if st["d"] == 0 else my - 1, N_DEV)
            r = rows(st, own)
            out_ref[r, :] = jnp.maximum(out_ref[r, :] * scale, 0.0)

        for st in streams:
            _sem_wait(st["cr"], 1)
            rs_desc(st, N_DEV - 3).wait_send()
            ag_stage_start(st, 0)

        def ag_iter(t, carry):
            for st in streams:
                ag_desc(st, t).wait_recv()
                slot = jnp.mod(t + N_DEV - 1, 2)
                r = rows(st, cr_ag(st, t))
                out_ref[r, :] = st["rb"][slot].astype(jnp.float32)
                _sem_signal(st["cr"], inc=1, device_id=(st["updev"],),
                            device_id_type=_DevId.MESH)

                @pl.when(t <= N_DEV - 3)
                def _(st=st, t=t):
                    _sem_wait(st["cr"], 1)
                    ag_desc(st, t - 1).wait_send()
                    ag_stage_start(st, t + 1)
            return carry

        lax.fori_loop(0, N_DEV - 1, ag_iter, 0)

        for st in streams:
            ag_desc(st, N_DEV - 3).wait_send()
            ag_desc(st, N_DEV - 2).wait_send()
            _sem_wait(st["cr"], 2)

    my_log = lax.axis_index("i")
    ring_of_log = jnp.asarray(RING_OF_LOG, dtype=jnp.int32)
    log_of_ring = jnp.asarray(LOG_OF_RING, dtype=jnp.int32)
    rp = ring_of_log[my_log]
    left_log = log_of_ring[jnp.mod(rp - 1, N_DEV)]
    right_log = log_of_ring[jnp.mod(rp + 1, N_DEV)]
    pos = jnp.stack([rp, left_log, right_log]).astype(jnp.int32)

    out_shape = jax.ShapeDtypeStruct((m, n), jnp.float32)
    return pl.pallas_call(
        body,
        out_shape=out_shape,
        in_specs=[
            pl.BlockSpec(memory_space=pltpu.VMEM),
            pl.BlockSpec(memory_space=pltpu.VMEM),
            pl.BlockSpec(memory_space=pltpu.SMEM),
            pl.BlockSpec(memory_space=pltpu.SMEM),
            pl.BlockSpec(memory_space=pltpu.SMEM),
        ],
        out_specs=pl.BlockSpec(memory_space=pltpu.VMEM),
        scratch_shapes=[
            pltpu.VMEM((2, sub, n), jnp.bfloat16),
            pltpu.VMEM((2, sub, n), jnp.bfloat16),
            pltpu.VMEM((2, sub, n), jnp.bfloat16),
            pltpu.VMEM((2, sub, n), jnp.bfloat16),
            pltpu.VMEM((2, sub, n), jnp.bfloat16),
            pltpu.VMEM((2, sub, n), jnp.bfloat16),
            pltpu.VMEM((2, sub, n), jnp.bfloat16),
            pltpu.VMEM((2, sub, n), jnp.bfloat16),
            pltpu.SemaphoreType.DMA((2,)),
            pltpu.SemaphoreType.DMA((2,)),
            pltpu.SemaphoreType.DMA((2,)),
            pltpu.SemaphoreType.DMA((2,)),
            pltpu.SemaphoreType.DMA((2,)),
            pltpu.SemaphoreType.DMA((2,)),
            pltpu.SemaphoreType.DMA((2,)),
            pltpu.SemaphoreType.DMA((2,)),
            pltpu.SemaphoreType.REGULAR,
            pltpu.SemaphoreType.REGULAR,
            pltpu.SemaphoreType.REGULAR,
            pltpu.SemaphoreType.REGULAR,
        ],
        compiler_params=pltpu.CompilerParams(
            collective_id=0, vmem_limit_bytes=64 * 1024 * 1024),
    )(x, w_mat, scale_x, scale_w, pos)
